# Initial kernel scaffold; baseline (speedup 1.0000x reference)
import jax
import jax.numpy as jnp
from jax.experimental import pallas as pl


def kernel(sentence, gazet, pos, words, char_table, pos_table):
    raise NotImplementedError("write your pallas kernel here")


# speedup vs baseline: 1.9319x; 1.9319x over previous
import functools

import jax
import jax.numpy as jnp
from jax import lax
from jax.experimental import pallas as pl
from jax.experimental.pallas import tpu as pltpu
from jax.experimental.pallas import tpu_sc as plsc

SEQ = 4096
CTX = 21
ROWS = SEQ * CTX
CHAR_D = 50
POS_D = 20
WORD_D = 100
GAZ_D = 15
EMB = CHAR_D + POS_D + WORD_D + GAZ_D

NC = 2
NS = 16
NW = NC * NS
TILE = 128
N_TILES = ROWS // TILE
TILES_PER_W = N_TILES // NW


def _sc_gather(sent2, pos2, char_table, pos_table):
    mesh = plsc.VectorSubcoreMesh(core_axis_name="c", subcore_axis_name="s")

    @functools.partial(
        pl.kernel,
        out_type=(
            jax.ShapeDtypeStruct((ROWS, CHAR_D), jnp.float32),
            jax.ShapeDtypeStruct((ROWS, POS_D), jnp.float32),
        ),
        mesh=mesh,
        scratch_types=[
            pltpu.VMEM((TILES_PER_W, TILE), jnp.int32),
            pltpu.VMEM((TILES_PER_W, TILE), jnp.int32),
            pltpu.VMEM_SHARED((1000, CHAR_D), jnp.float32),
            pltpu.VMEM_SHARED((627, POS_D), jnp.float32),
            pltpu.VMEM((TILE, CHAR_D), jnp.float32),
            pltpu.VMEM((TILE, POS_D), jnp.float32),
            pltpu.SemaphoreType.DMA,
        ],
    )
    def k(sent_hbm, pos_hbm, ctab_hbm, ptab_hbm, gc_hbm, gp_hbm,
          idx_c, idx_p, ctab_v, ptab_v, bufc, bufp, sem):
        wid = lax.axis_index("s") * NC + lax.axis_index("c")
        t0 = wid * TILES_PER_W
        @pl.when(lax.axis_index("s") == 0)
        def _():
            pltpu.sync_copy(ctab_hbm, ctab_v)
            pltpu.sync_copy(ptab_hbm, ptab_v)

        pltpu.sync_copy(sent_hbm.at[wid], idx_c)
        pltpu.sync_copy(pos_hbm.at[wid], idx_p)
        plsc.subcore_barrier()

        @pl.loop(0, TILES_PER_W)
        def _(j):
            r0 = (t0 + j) * TILE
            cc = pltpu.async_copy(ctab_v.at[idx_c.at[j]], bufc, sem)
            cp = pltpu.async_copy(ptab_v.at[idx_p.at[j]], bufp, sem)
            cc.wait()
            cp.wait()
            pltpu.sync_copy(bufc, gc_hbm.at[pl.ds(r0, TILE)])
            pltpu.sync_copy(bufp, gp_hbm.at[pl.ds(r0, TILE)])

    return k(sent2, pos2, char_table, pos_table)


def _assemble_body(gc_ref, gp_ref, w_ref, z_ref, pe_ref, out_ref):
    pe = pe_ref[...]
    out_ref[...] = jnp.concatenate(
        [
            gc_ref[...] + pe[:, 0:CHAR_D],
            gp_ref[...] + pe[:, CHAR_D:CHAR_D + POS_D],
            w_ref[...] + pe[:, CHAR_D + POS_D:CHAR_D + POS_D + WORD_D],
            z_ref[...] + pe[:, CHAR_D + POS_D + WORD_D:EMB],
        ],
        axis=1,
    )


def _tc_assemble(gc, gp, w2, z2, pe_rep):
    bm = pe_rep.shape[0]
    grid = ROWS // bm
    return pl.pallas_call(
        _assemble_body,
        grid=(grid,),
        in_specs=[
            pl.BlockSpec((bm, CHAR_D), lambda i: (i, 0)),
            pl.BlockSpec((bm, POS_D), lambda i: (i, 0)),
            pl.BlockSpec((bm, WORD_D), lambda i: (i, 0)),
            pl.BlockSpec((bm, GAZ_D), lambda i: (i, 0)),
            pl.BlockSpec((bm, EMB), lambda i: (0, 0)),
        ],
        out_specs=pl.BlockSpec((bm, EMB), lambda i: (i, 0)),
        out_shape=jax.ShapeDtypeStruct((ROWS, EMB), jnp.float32),
    )(gc, gp, w2, z2, pe_rep)


def kernel(sentence, gazet, pos, words, char_table, pos_table):
    sent2 = sentence.reshape(NW, TILES_PER_W, TILE).astype(jnp.int32)
    pos2 = pos.reshape(NW, TILES_PER_W, TILE).astype(jnp.int32)
    gc, gp = _sc_gather(sent2, pos2, char_table, pos_table)

    j = jnp.arange(1, CTX + 1, dtype=jnp.float32)[:, None]
    k = jnp.arange(1, EMB + 1, dtype=jnp.float32)[None, :]
    pe = 1.0 - j / CTX - (k / EMB) * (1.0 - 2.0 * j / CTX)
    pe_rep = jnp.tile(pe, (TILE, 1))

    out2 = _tc_assemble(
        gc, gp, words.reshape(ROWS, WORD_D), gazet.reshape(ROWS, GAZ_D), pe_rep
    )
    return out2.reshape(SEQ, CTX, EMB)


# speedup vs baseline: 2.8126x; 1.4558x over previous
import functools

import jax
import jax.numpy as jnp
from jax import lax
from jax.experimental import pallas as pl
from jax.experimental.pallas import tpu as pltpu
from jax.experimental.pallas import tpu_sc as plsc

SEQ = 4096
CTX = 21
CTX_PAD = 24
CHAR_V = 1000
POS_V = 627
CHAR_D = 50
POS_D = 20
WORD_D = 100
GAZ_D = 15
EMB = CHAR_D + POS_D + WORD_D + GAZ_D

NC = 2
NS = 16
NW = NC * NS
SEQ_PER_W = SEQ // NW
CHUNK = 8
N_CHUNKS = SEQ_PER_W // CHUNK
ROWS_PAD = SEQ * CTX_PAD


def _sc_gather(sentence, pos, char_table, pos_table):
    mesh = plsc.VectorSubcoreMesh(core_axis_name="c", subcore_axis_name="s")

    @functools.partial(
        pl.kernel,
        out_type=(
            jax.ShapeDtypeStruct((ROWS_PAD, CHAR_D), jnp.float32),
            jax.ShapeDtypeStruct((ROWS_PAD, POS_D), jnp.float32),
        ),
        mesh=mesh,
        scratch_types=[
            pltpu.VMEM((SEQ_PER_W, CTX), jnp.int32),
            pltpu.VMEM((SEQ_PER_W, CTX), jnp.int32),
            pltpu.VMEM_SHARED((CHAR_V, CHAR_D), jnp.float32),
            pltpu.VMEM_SHARED((POS_V, POS_D), jnp.float32),
            pltpu.VMEM((CHUNK * CTX_PAD, CHAR_D), jnp.float32),
            pltpu.VMEM((CHUNK * CTX_PAD, POS_D), jnp.float32),
            pltpu.SemaphoreType.DMA,
        ],
    )
    def k(sent_hbm, pos_hbm, ctab_hbm, ptab_hbm, gc_hbm, gp_hbm,
          idx_c, idx_p, ctab_sh, ptab_sh, bufc, bufp, sem):
        wid = lax.axis_index("s") * NC + lax.axis_index("c")
        s0 = wid * SEQ_PER_W
        @pl.when(lax.axis_index("s") == 0)
        def _():
            pltpu.sync_copy(ctab_hbm, ctab_sh)
            pltpu.sync_copy(ptab_hbm, ptab_sh)

        pltpu.sync_copy(sent_hbm.at[pl.ds(s0, SEQ_PER_W)], idx_c)
        pltpu.sync_copy(pos_hbm.at[pl.ds(s0, SEQ_PER_W)], idx_p)
        plsc.subcore_barrier()

        @pl.loop(0, N_CHUNKS)
        def _(c):
            @pl.loop(0, CHUNK)
            def _(s):
                g = c * CHUNK + s
                pltpu.async_copy(
                    ctab_sh.at[idx_c.at[g]],
                    bufc.at[pl.ds(s * CTX_PAD, CTX)], sem)
                pltpu.async_copy(
                    ptab_sh.at[idx_p.at[g]],
                    bufp.at[pl.ds(s * CTX_PAD, CTX)], sem)

            @pl.loop(0, CHUNK)
            def _(s):
                g = c * CHUNK + s
                pltpu.make_async_copy(
                    ctab_sh.at[idx_c.at[g]],
                    bufc.at[pl.ds(s * CTX_PAD, CTX)], sem).wait()
                pltpu.make_async_copy(
                    ptab_sh.at[idx_p.at[g]],
                    bufp.at[pl.ds(s * CTX_PAD, CTX)], sem).wait()

            r0 = (s0 + c * CHUNK) * CTX_PAD
            pltpu.sync_copy(bufc, gc_hbm.at[pl.ds(r0, CHUNK * CTX_PAD)])
            pltpu.sync_copy(bufp, gp_hbm.at[pl.ds(r0, CHUNK * CTX_PAD)])

    return k(sentence, pos, char_table, pos_table)


BS = 128


def _assemble_body(gc_ref, gp_ref, w_ref, z_ref, pe_ref, out_ref):
    pe = pe_ref[...][None]
    gc = gc_ref[:, 0:CTX, :]
    gp = gp_ref[:, 0:CTX, :]
    out_ref[...] = jnp.concatenate(
        [
            gc + pe[:, :, 0:CHAR_D],
            gp + pe[:, :, CHAR_D:CHAR_D + POS_D],
            w_ref[...] + pe[:, :, CHAR_D + POS_D:CHAR_D + POS_D + WORD_D],
            z_ref[...] + pe[:, :, CHAR_D + POS_D + WORD_D:EMB],
        ],
        axis=2,
    )


def _tc_assemble(gc3, gp3, words, gazet, pe):
    grid = SEQ // BS
    return pl.pallas_call(
        _assemble_body,
        grid=(grid,),
        in_specs=[
            pl.BlockSpec((BS, CTX_PAD, CHAR_D), lambda i: (i, 0, 0)),
            pl.BlockSpec((BS, CTX_PAD, POS_D), lambda i: (i, 0, 0)),
            pl.BlockSpec((BS, CTX, WORD_D), lambda i: (i, 0, 0)),
            pl.BlockSpec((BS, CTX, GAZ_D), lambda i: (i, 0, 0)),
            pl.BlockSpec((CTX, EMB), lambda i: (0, 0)),
        ],
        out_specs=pl.BlockSpec((BS, CTX, EMB), lambda i: (i, 0, 0)),
        out_shape=jax.ShapeDtypeStruct((SEQ, CTX, EMB), jnp.float32),
    )(gc3, gp3, words, gazet, pe)


def kernel(sentence, gazet, pos, words, char_table, pos_table):
    gc2, gp2 = _sc_gather(sentence.astype(jnp.int32), pos.astype(jnp.int32),
                          char_table, pos_table)
    gc3 = gc2.reshape(SEQ, CTX_PAD, CHAR_D)
    gp3 = gp2.reshape(SEQ, CTX_PAD, POS_D)

    j = jnp.arange(1, CTX + 1, dtype=jnp.float32)[:, None]
    k = jnp.arange(1, EMB + 1, dtype=jnp.float32)[None, :]
    pe = 1.0 - j / CTX - (k / EMB) * (1.0 - 2.0 * j / CTX)

    return _tc_assemble(gc3, gp3, words, gazet, pe)


# speedup vs baseline: 2.8243x; 1.0042x over previous
import functools

import jax
import jax.numpy as jnp
from jax import lax
from jax.experimental import pallas as pl
from jax.experimental.pallas import tpu as pltpu
from jax.experimental.pallas import tpu_sc as plsc

SEQ = 4096
CTX = 21
CTX_PAD = 24
CHAR_V = 1000
POS_V = 627
CHAR_D = 50
POS_D = 20
WORD_D = 100
GAZ_D = 15
EMB = CHAR_D + POS_D + WORD_D + GAZ_D

NC = 2
NS = 16
NW = NC * NS
SEQ_PER_W = SEQ // NW
CHUNK = 8
N_CHUNKS = SEQ_PER_W // CHUNK
ROWS_PAD = SEQ * CTX_PAD


def _sc_gather(sentence, pos, char_table, pos_table):
    mesh = plsc.VectorSubcoreMesh(core_axis_name="c", subcore_axis_name="s")

    @functools.partial(
        pl.kernel,
        out_type=(
            jax.ShapeDtypeStruct((SEQ, CTX_PAD, CHAR_D), jnp.float32),
            jax.ShapeDtypeStruct((SEQ, CTX_PAD, POS_D), jnp.float32),
        ),
        mesh=mesh,
        scratch_types=[
            pltpu.VMEM((SEQ_PER_W, CTX), jnp.int32),
            pltpu.VMEM((SEQ_PER_W, CTX), jnp.int32),
            pltpu.VMEM_SHARED((CHAR_V, CHAR_D), jnp.float32),
            pltpu.VMEM_SHARED((POS_V, POS_D), jnp.float32),
            pltpu.VMEM((CHUNK * CTX_PAD, CHAR_D), jnp.float32),
            pltpu.VMEM((CHUNK * CTX_PAD, POS_D), jnp.float32),
            pltpu.SemaphoreType.DMA,
        ],
    )
    def k(sent_hbm, pos_hbm, ctab_hbm, ptab_hbm, gc_hbm, gp_hbm,
          idx_c, idx_p, ctab_sh, ptab_sh, bufc, bufp, sem):
        wid = lax.axis_index("s") * NC + lax.axis_index("c")
        s0 = wid * SEQ_PER_W
        @pl.when(lax.axis_index("s") == 0)
        def _():
            pltpu.sync_copy(ctab_hbm, ctab_sh)
            pltpu.sync_copy(ptab_hbm, ptab_sh)

        pltpu.sync_copy(sent_hbm.at[pl.ds(s0, SEQ_PER_W)], idx_c)
        pltpu.sync_copy(pos_hbm.at[pl.ds(s0, SEQ_PER_W)], idx_p)
        plsc.subcore_barrier()

        @pl.loop(0, N_CHUNKS)
        def _(c):
            @pl.loop(0, CHUNK)
            def _(s):
                g = c * CHUNK + s
                pltpu.async_copy(
                    ctab_sh.at[idx_c.at[g]],
                    bufc.at[pl.ds(s * CTX_PAD, CTX)], sem)
                pltpu.async_copy(
                    ptab_sh.at[idx_p.at[g]],
                    bufp.at[pl.ds(s * CTX_PAD, CTX)], sem)

            @pl.loop(0, CHUNK)
            def _(s):
                g = c * CHUNK + s
                pltpu.make_async_copy(
                    ctab_sh.at[idx_c.at[g]],
                    bufc.at[pl.ds(s * CTX_PAD, CTX)], sem).wait()
                pltpu.make_async_copy(
                    ptab_sh.at[idx_p.at[g]],
                    bufp.at[pl.ds(s * CTX_PAD, CTX)], sem).wait()

            s_off = s0 + c * CHUNK
            pltpu.sync_copy(bufc.reshape(CHUNK, CTX_PAD, CHAR_D),
                            gc_hbm.at[pl.ds(s_off, CHUNK)])
            pltpu.sync_copy(bufp.reshape(CHUNK, CTX_PAD, POS_D),
                            gp_hbm.at[pl.ds(s_off, CHUNK)])

    return k(sentence, pos, char_table, pos_table)


BS = 128


def _assemble_body(gc_ref, gp_ref, w_ref, z_ref, pe_ref, out_ref):
    pe = pe_ref[...][None]
    gc = gc_ref[:, 0:CTX, :]
    gp = gp_ref[:, 0:CTX, :]
    out_ref[...] = jnp.concatenate(
        [
            gc + pe[:, :, 0:CHAR_D],
            gp + pe[:, :, CHAR_D:CHAR_D + POS_D],
            w_ref[...] + pe[:, :, CHAR_D + POS_D:CHAR_D + POS_D + WORD_D],
            z_ref[...] + pe[:, :, CHAR_D + POS_D + WORD_D:EMB],
        ],
        axis=2,
    )


def _tc_assemble(gc3, gp3, words, gazet, pe):
    grid = SEQ // BS
    return pl.pallas_call(
        _assemble_body,
        grid=(grid,),
        in_specs=[
            pl.BlockSpec((BS, CTX_PAD, CHAR_D), lambda i: (i, 0, 0)),
            pl.BlockSpec((BS, CTX_PAD, POS_D), lambda i: (i, 0, 0)),
            pl.BlockSpec((BS, CTX, WORD_D), lambda i: (i, 0, 0)),
            pl.BlockSpec((BS, CTX, GAZ_D), lambda i: (i, 0, 0)),
            pl.BlockSpec((CTX, EMB), lambda i: (0, 0)),
        ],
        out_specs=pl.BlockSpec((BS, CTX, EMB), lambda i: (i, 0, 0)),
        out_shape=jax.ShapeDtypeStruct((SEQ, CTX, EMB), jnp.float32),
    )(gc3, gp3, words, gazet, pe)


def kernel(sentence, gazet, pos, words, char_table, pos_table):
    gc3, gp3 = _sc_gather(sentence.astype(jnp.int32), pos.astype(jnp.int32),
                          char_table, pos_table)

    j = jnp.arange(1, CTX + 1, dtype=jnp.float32)[:, None]
    k = jnp.arange(1, EMB + 1, dtype=jnp.float32)[None, :]
    pe = 1.0 - j / CTX - (k / EMB) * (1.0 - 2.0 * j / CTX)

    return _tc_assemble(gc3, gp3, words, gazet, pe)


# speedup vs baseline: 3.7699x; 1.3348x over previous
import functools

import jax
import jax.numpy as jnp
from jax import lax
from jax.experimental import pallas as pl
from jax.experimental.pallas import tpu as pltpu
from jax.experimental.pallas import tpu_sc as plsc

SEQ = 4096
CTX = 21
CTX_PAD = 24
CHAR_V = 1000
POS_V = 627
CHAR_D = 50
POS_D = 20
WORD_D = 100
GAZ_D = 15
EMB = CHAR_D + POS_D + WORD_D + GAZ_D

NC = 2
NS = 16
NW = NC * NS
SEQ_PER_W = SEQ // NW
CHUNK = 8
N_CHUNKS = SEQ_PER_W // CHUNK
ROWS_PAD = SEQ * CTX_PAD


def _sc_gather(sentence, pos, char_table, pos_table):
    mesh = plsc.VectorSubcoreMesh(core_axis_name="c", subcore_axis_name="s")

    @functools.partial(
        pl.kernel,
        out_type=(
            jax.ShapeDtypeStruct((ROWS_PAD, CHAR_D), jnp.float32),
            jax.ShapeDtypeStruct((ROWS_PAD, POS_D), jnp.float32),
        ),
        mesh=mesh,
        scratch_types=[
            pltpu.VMEM((SEQ_PER_W, CTX), jnp.int32),
            pltpu.VMEM((SEQ_PER_W, CTX), jnp.int32),
            pltpu.VMEM_SHARED((CHAR_V, CHAR_D), jnp.float32),
            pltpu.VMEM_SHARED((POS_V, POS_D), jnp.float32),
            pltpu.VMEM((CHUNK * CTX_PAD, CHAR_D), jnp.float32),
            pltpu.VMEM((CHUNK * CTX_PAD, POS_D), jnp.float32),
            pltpu.SemaphoreType.DMA,
        ],
    )
    def k(sent_hbm, pos_hbm, ctab_hbm, ptab_hbm, gc_hbm, gp_hbm,
          idx_c, idx_p, ctab_sh, ptab_sh, bufc, bufp, sem):
        wid = lax.axis_index("s") * NC + lax.axis_index("c")
        s0 = wid * SEQ_PER_W
        @pl.when(lax.axis_index("s") == 0)
        def _():
            pltpu.sync_copy(ctab_hbm, ctab_sh)
            pltpu.sync_copy(ptab_hbm, ptab_sh)

        pltpu.sync_copy(sent_hbm.at[pl.ds(s0, SEQ_PER_W)], idx_c)
        pltpu.sync_copy(pos_hbm.at[pl.ds(s0, SEQ_PER_W)], idx_p)
        plsc.subcore_barrier()

        @pl.loop(0, N_CHUNKS)
        def _(c):
            @pl.loop(0, CHUNK)
            def _(s):
                g = c * CHUNK + s
                pltpu.async_copy(
                    ctab_sh.at[idx_c.at[g]],
                    bufc.at[pl.ds(s * CTX_PAD, CTX)], sem)
                pltpu.async_copy(
                    ptab_sh.at[idx_p.at[g]],
                    bufp.at[pl.ds(s * CTX_PAD, CTX)], sem)

            @pl.loop(0, CHUNK)
            def _(s):
                g = c * CHUNK + s
                pltpu.make_async_copy(
                    ctab_sh.at[idx_c.at[g]],
                    bufc.at[pl.ds(s * CTX_PAD, CTX)], sem).wait()
                pltpu.make_async_copy(
                    ptab_sh.at[idx_p.at[g]],
                    bufp.at[pl.ds(s * CTX_PAD, CTX)], sem).wait()

            r0 = (s0 + c * CHUNK) * CTX_PAD
            pltpu.sync_copy(bufc, gc_hbm.at[pl.ds(r0, CHUNK * CTX_PAD)])
            pltpu.sync_copy(bufp, gp_hbm.at[pl.ds(r0, CHUNK * CTX_PAD)])

    return k(sentence, pos, char_table, pos_table)


def kernel(sentence, gazet, pos, words, char_table, pos_table):
    gc2, gp2 = _sc_gather(sentence.astype(jnp.int32), pos.astype(jnp.int32),
                          char_table, pos_table)
    gc3 = gc2.reshape(SEQ, CTX_PAD, CHAR_D)[:, :CTX, :]
    gp3 = gp2.reshape(SEQ, CTX_PAD, POS_D)[:, :CTX, :]

    j = jnp.arange(1, CTX + 1, dtype=jnp.float32)[:, None]
    k = jnp.arange(1, EMB + 1, dtype=jnp.float32)[None, :]
    pe = 1.0 - j / CTX - (k / EMB) * (1.0 - 2.0 * j / CTX)

    return jnp.concatenate(
        [
            gc3 + pe[:, 0:CHAR_D],
            gp3 + pe[:, CHAR_D:CHAR_D + POS_D],
            words + pe[:, CHAR_D + POS_D:CHAR_D + POS_D + WORD_D],
            gazet + pe[:, CHAR_D + POS_D + WORD_D:EMB],
        ],
        axis=2,
    )
